# baseline (speedup 1.0000x reference)
import functools

import jax
import jax.numpy as jnp
from jax import lax
from jax.experimental import pallas as pl
from jax.experimental.pallas import tpu as pltpu
from jax.experimental.pallas import tpu_sc as plsc


def kernel(action_type, action_embeddings):
    (B,) = action_type.shape
    V, D = action_embeddings.shape

    info = plsc.get_sparse_core_info()
    NC, NS = info.num_cores, info.num_subcores
    NW = NC * NS
    b_per_w = B // NW

    mesh = plsc.VectorSubcoreMesh(core_axis_name="c", subcore_axis_name="s")

    @functools.partial(
        pl.kernel,
        mesh=mesh,
        out_type=jax.ShapeDtypeStruct((B, D), jnp.float32),
        scratch_types=[
            pltpu.VMEM((b_per_w,), jnp.int32),
            pltpu.SemaphoreType.DMA,
        ],
    )
    def gather_kernel(idx_hbm, table_hbm, out_hbm, idx_v, dsem):
        wid = lax.axis_index("s") * NC + lax.axis_index("c")
        base = wid * b_per_w
        pltpu.sync_copy(idx_hbm.at[pl.ds(base, b_per_w)], idx_v)

        def fire(g, carry):
            iv = idx_v[pl.ds(g * 16, 16)]
            for k in range(16):
                row = iv[k]
                pltpu.async_copy(
                    table_hbm.at[row], out_hbm.at[base + g * 16 + k], dsem
                )
            return carry

        lax.fori_loop(0, b_per_w // 16, fire, 0)

        def drain(g, carry):
            for k in range(16):
                pltpu.make_async_copy(
                    table_hbm.at[0], out_hbm.at[base], dsem
                ).wait()
            return carry

        lax.fori_loop(0, b_per_w // 16, drain, 0)

    return gather_kernel(action_type.astype(jnp.int32), action_embeddings)


# speedup vs baseline: 19.4691x; 19.4691x over previous
import functools

import jax
import jax.numpy as jnp
from jax import lax
from jax.experimental import pallas as pl
from jax.experimental.pallas import tpu as pltpu
from jax.experimental.pallas import tpu_sc as plsc


def kernel(action_type, action_embeddings):
    (B,) = action_type.shape
    V, D = action_embeddings.shape

    info = plsc.get_sparse_core_info()
    NC, NS = info.num_cores, info.num_subcores
    NW = NC * NS
    b_per_w = B // NW

    mesh = plsc.VectorSubcoreMesh(core_axis_name="c", subcore_axis_name="s")

    @functools.partial(
        pl.kernel,
        mesh=mesh,
        out_type=jax.ShapeDtypeStruct((B, D), jnp.float32),
        scratch_types=[
            pltpu.VMEM_SHARED((V, D), jnp.float32),
            pltpu.VMEM((V, D), jnp.float32),
            pltpu.VMEM((b_per_w,), jnp.int32),
            pltpu.SemaphoreType.DMA,
            pltpu.SemaphoreType.DMA,
        ],
    )
    def gather_kernel(
        idx_hbm, table_hbm, out_hbm, table_sh, table_v, idx_v, ssem, dsem
    ):
        sid = lax.axis_index("s")
        wid = sid * NC + lax.axis_index("c")
        base = wid * b_per_w

        icopy = pltpu.async_copy(idx_hbm.at[pl.ds(base, b_per_w)], idx_v, ssem)

        @pl.when(sid == 0)
        def _():
            pltpu.async_copy(table_hbm, table_sh, ssem).wait()

        plsc.subcore_barrier()
        pltpu.async_copy(table_sh, table_v, ssem).wait()
        icopy.wait()

        def fire(g, carry):
            iv = idx_v[pl.ds(g * 16, 16)]
            for k in range(16):
                row = iv[k]
                pltpu.async_copy(
                    table_v.at[row], out_hbm.at[base + g * 16 + k], dsem
                )
            return carry

        lax.fori_loop(0, b_per_w // 16, fire, 0)

        def drain(g, carry):
            for k in range(16):
                pltpu.make_async_copy(
                    table_v.at[0], out_hbm.at[base], dsem
                ).wait()
            return carry

        lax.fori_loop(0, b_per_w // 16, drain, 0)

    return gather_kernel(action_type.astype(jnp.int32), action_embeddings)
